# baseline (device time: 21799 ns/iter reference)
import jax
import jax.numpy as jnp
from jax import lax
from jax.experimental import pallas as pl
from jax.experimental.pallas import tpu as pltpu


def kernel(dy, W):
    m, k = dy.shape
    n, k2 = W.shape
    assert k == k2

    def body(dy_ref, w_ref, out_ref, peer_ref, send_sem, recv_sem):
        my_x = lax.axis_index("x")
        my_y = lax.axis_index("y")
        nbr = (my_x, 1 - my_y)

        barrier_sem = pltpu.get_barrier_semaphore()
        pl.semaphore_signal(
            barrier_sem, inc=1, device_id=nbr,
            device_id_type=pl.DeviceIdType.MESH,
        )
        pl.semaphore_wait(barrier_sem, 1)

        partial = lax.dot_general(
            dy_ref[...].astype(jnp.bfloat16),
            w_ref[...].astype(jnp.bfloat16),
            dimension_numbers=(((1,), (1,)), ((), ())),
            preferred_element_type=jnp.float32,
        )
        out_ref[...] = partial

        rdma = pltpu.make_async_remote_copy(
            src_ref=out_ref,
            dst_ref=peer_ref,
            send_sem=send_sem,
            recv_sem=recv_sem,
            device_id=nbr,
            device_id_type=pl.DeviceIdType.MESH,
        )
        rdma.start()
        rdma.wait()
        out_ref[...] = out_ref[...] + peer_ref[...]

    return pl.pallas_call(
        body,
        out_shape=jax.ShapeDtypeStruct((m, n), jnp.float32),
        in_specs=[
            pl.BlockSpec(memory_space=pltpu.VMEM),
            pl.BlockSpec(memory_space=pltpu.VMEM),
        ],
        out_specs=pl.BlockSpec(memory_space=pltpu.VMEM),
        scratch_shapes=[
            pltpu.VMEM((m, n), jnp.float32),
            pltpu.SemaphoreType.DMA,
            pltpu.SemaphoreType.DMA,
        ],
        compiler_params=pltpu.CompilerParams(collective_id=0),
    )(dy, W)


# device time: 16665 ns/iter; 1.3081x vs baseline; 1.3081x over previous
import jax
import jax.numpy as jnp
from jax import lax
from jax.experimental import pallas as pl
from jax.experimental.pallas import tpu as pltpu

C = 8


def kernel(dy, W):
    m, k = dy.shape
    n, k2 = W.shape
    assert k == k2
    half = m // 2
    ch = half // C

    def body(dy_ref, w_ref, out_ref, pbuf, ybuf, xbuf,
             ysend_sems, yrecv_sems, xsend_sems, xrecv_sems):
        my_x = lax.axis_index("x")
        my_y = lax.axis_index("y")
        ynbr = (my_x, 1 - my_y)
        xnbr = (1 - my_x, my_y)
        row0 = my_x * half
        other0 = (1 - my_x) * half

        barrier_sem = pltpu.get_barrier_semaphore()
        for nbr in (ynbr, xnbr):
            pl.semaphore_signal(
                barrier_sem, inc=1, device_id=nbr,
                device_id_type=pl.DeviceIdType.MESH,
            )
        pl.semaphore_wait(barrier_sem, 2)

        def matmul(rows):
            return lax.dot_general(
                dy_ref[rows, :].astype(jnp.bfloat16),
                w_ref[...].astype(jnp.bfloat16),
                dimension_numbers=(((1,), (1,)), ((), ())),
                preferred_element_type=jnp.float32,
            )

        pm = matmul(pl.ds(row0, half))
        pbuf[...] = pm.astype(jnp.bfloat16)

        y_rdmas = []
        for c in range(C):
            r = pl.ds(c * ch, ch)
            rdma = pltpu.make_async_remote_copy(
                src_ref=pbuf.at[r, :],
                dst_ref=ybuf.at[r, :],
                send_sem=ysend_sems.at[c],
                recv_sem=yrecv_sems.at[c],
                device_id=ynbr,
                device_id_type=pl.DeviceIdType.MESH,
            )
            rdma.start()
            y_rdmas.append(rdma)

        po = matmul(pl.ds(other0, half))

        x_rdmas = []
        for c in range(C):
            y_rdmas[c].wait_recv()
            r = pl.ds(c * ch, ch)
            rdma = pltpu.make_async_remote_copy(
                src_ref=ybuf.at[r, :],
                dst_ref=xbuf.at[r, :],
                send_sem=xsend_sems.at[c],
                recv_sem=xrecv_sems.at[c],
                device_id=xnbr,
                device_id_type=pl.DeviceIdType.MESH,
            )
            rdma.start()
            x_rdmas.append(rdma)

        out_ref[pl.ds(row0, half), :] = pm + ybuf[...].astype(jnp.float32)

        for c in range(C):
            x_rdmas[c].wait_recv()
        out_ref[pl.ds(other0, half), :] = po + xbuf[...].astype(jnp.float32)

        for c in range(C):
            y_rdmas[c].wait_send()
            x_rdmas[c].wait_send()

    return pl.pallas_call(
        body,
        out_shape=jax.ShapeDtypeStruct((m, n), jnp.float32),
        in_specs=[
            pl.BlockSpec(memory_space=pltpu.VMEM),
            pl.BlockSpec(memory_space=pltpu.VMEM),
        ],
        out_specs=pl.BlockSpec(memory_space=pltpu.VMEM),
        scratch_shapes=[
            pltpu.VMEM((half, n), jnp.bfloat16),
            pltpu.VMEM((half, n), jnp.bfloat16),
            pltpu.VMEM((half, n), jnp.bfloat16),
            pltpu.SemaphoreType.DMA((C,)),
            pltpu.SemaphoreType.DMA((C,)),
            pltpu.SemaphoreType.DMA((C,)),
            pltpu.SemaphoreType.DMA((C,)),
        ],
        compiler_params=pltpu.CompilerParams(collective_id=0),
    )(dy, W)


# device time: 16504 ns/iter; 1.3208x vs baseline; 1.0098x over previous
import jax
import jax.numpy as jnp
from jax import lax
from jax.experimental import pallas as pl
from jax.experimental.pallas import tpu as pltpu

C = 8


def kernel(dy, W):
    m, k = dy.shape
    n, k2 = W.shape
    assert k == k2
    half = m // 2
    ch = half // C

    def body(dy_ref, w_ref, out_ref, pbuf, ybuf, xbuf,
             ysend_sems, yrecv_sems, xsend_sems, xrecv_sems):
        my_x = lax.axis_index("x")
        my_y = lax.axis_index("y")
        ynbr = (my_x, 1 - my_y)
        xnbr = (1 - my_x, my_y)
        row0 = my_x * half
        other0 = (1 - my_x) * half

        wbf = w_ref[...].astype(jnp.bfloat16)
        dybf = dy_ref[...].astype(jnp.bfloat16)
        p = lax.dot_general(
            dybf, wbf,
            dimension_numbers=(((1,), (1,)), ((), ())),
            preferred_element_type=jnp.float32,
        )
        out_ref[...] = p
        pbuf[...] = out_ref[pl.ds(row0, half), :].astype(jnp.bfloat16)

        barrier_sem = pltpu.get_barrier_semaphore()
        for nbr in (ynbr, xnbr):
            pl.semaphore_signal(
                barrier_sem, inc=1, device_id=nbr,
                device_id_type=pl.DeviceIdType.MESH,
            )
        pl.semaphore_wait(barrier_sem, 2)

        y_rdmas = []
        for c in range(C):
            r = pl.ds(c * ch, ch)
            rdma = pltpu.make_async_remote_copy(
                src_ref=pbuf.at[r, :],
                dst_ref=ybuf.at[r, :],
                send_sem=ysend_sems.at[c],
                recv_sem=yrecv_sems.at[c],
                device_id=ynbr,
                device_id_type=pl.DeviceIdType.MESH,
            )
            rdma.start()
            y_rdmas.append(rdma)

        x_rdmas = []
        for c in range(C):
            y_rdmas[c].wait_recv()
            r = pl.ds(c * ch, ch)
            rdma = pltpu.make_async_remote_copy(
                src_ref=ybuf.at[r, :],
                dst_ref=xbuf.at[r, :],
                send_sem=xsend_sems.at[c],
                recv_sem=xrecv_sems.at[c],
                device_id=xnbr,
                device_id_type=pl.DeviceIdType.MESH,
            )
            rdma.start()
            x_rdmas.append(rdma)

        out_ref[pl.ds(row0, half), :] = (
            out_ref[pl.ds(row0, half), :] + ybuf[...].astype(jnp.float32)
        )

        for c in range(C):
            x_rdmas[c].wait_recv()
        out_ref[pl.ds(other0, half), :] = (
            out_ref[pl.ds(other0, half), :] + xbuf[...].astype(jnp.float32)
        )

        for c in range(C):
            y_rdmas[c].wait_send()
            x_rdmas[c].wait_send()

    return pl.pallas_call(
        body,
        out_shape=jax.ShapeDtypeStruct((m, n), jnp.float32),
        in_specs=[
            pl.BlockSpec(memory_space=pltpu.VMEM),
            pl.BlockSpec(memory_space=pltpu.VMEM),
        ],
        out_specs=pl.BlockSpec(memory_space=pltpu.VMEM),
        scratch_shapes=[
            pltpu.VMEM((half, n), jnp.bfloat16),
            pltpu.VMEM((half, n), jnp.bfloat16),
            pltpu.VMEM((half, n), jnp.bfloat16),
            pltpu.SemaphoreType.DMA((C,)),
            pltpu.SemaphoreType.DMA((C,)),
            pltpu.SemaphoreType.DMA((C,)),
            pltpu.SemaphoreType.DMA((C,)),
        ],
        compiler_params=pltpu.CompilerParams(collective_id=0),
    )(dy, W)


# device time: 15819 ns/iter; 1.3780x vs baseline; 1.0433x over previous
import jax
import jax.numpy as jnp
from jax import lax
from jax.experimental import pallas as pl
from jax.experimental.pallas import tpu as pltpu

C = 8


def kernel(dy, W):
    m, k = dy.shape
    n, k2 = W.shape
    assert k == k2
    half = m // 2
    ch = half // C

    def body(dy_ref, w_ref, out_ref, pbuf, ybuf, sbuf, xbuf,
             ysend_sems, yrecv_sems, xsend_sems, xrecv_sems):
        my_x = lax.axis_index("x")
        my_y = lax.axis_index("y")
        ynbr = (my_x, 1 - my_y)
        xnbr = (1 - my_x, my_y)
        row0 = my_x * half
        other0 = (1 - my_x) * half

        wbf = w_ref[...].astype(jnp.bfloat16)
        dybf = dy_ref[pl.ds(row0, half), :].astype(jnp.bfloat16)
        pm = lax.dot_general(
            dybf, wbf,
            dimension_numbers=(((1,), (1,)), ((), ())),
            preferred_element_type=jnp.float32,
        )
        pbuf[...] = pm.astype(jnp.bfloat16)

        barrier_sem = pltpu.get_barrier_semaphore()
        for nbr in (ynbr, xnbr):
            pl.semaphore_signal(
                barrier_sem, inc=1, device_id=nbr,
                device_id_type=pl.DeviceIdType.MESH,
            )
        pl.semaphore_wait(barrier_sem, 2)

        y_rdmas = []
        for c in range(C):
            r = pl.ds(c * ch, ch)
            rdma = pltpu.make_async_remote_copy(
                src_ref=pbuf.at[r, :],
                dst_ref=ybuf.at[r, :],
                send_sem=ysend_sems.at[c],
                recv_sem=yrecv_sems.at[c],
                device_id=ynbr,
                device_id_type=pl.DeviceIdType.MESH,
            )
            rdma.start()
            y_rdmas.append(rdma)

        x_rdmas = []
        for c in range(C):
            y_rdmas[c].wait_recv()
            r = pl.ds(c * ch, ch)
            sum_c = pm[c * ch:(c + 1) * ch, :] + ybuf[r, :].astype(jnp.float32)
            out_ref[pl.ds(row0 + c * ch, ch), :] = sum_c
            sbuf[r, :] = sum_c.astype(jnp.bfloat16)
            rdma = pltpu.make_async_remote_copy(
                src_ref=sbuf.at[r, :],
                dst_ref=xbuf.at[r, :],
                send_sem=xsend_sems.at[c],
                recv_sem=xrecv_sems.at[c],
                device_id=xnbr,
                device_id_type=pl.DeviceIdType.MESH,
            )
            rdma.start()
            x_rdmas.append(rdma)

        for c in range(C):
            x_rdmas[c].wait_recv()
        out_ref[pl.ds(other0, half), :] = xbuf[...].astype(jnp.float32)

        for c in range(C):
            y_rdmas[c].wait_send()
            x_rdmas[c].wait_send()

    return pl.pallas_call(
        body,
        out_shape=jax.ShapeDtypeStruct((m, n), jnp.float32),
        in_specs=[
            pl.BlockSpec(memory_space=pltpu.VMEM),
            pl.BlockSpec(memory_space=pltpu.VMEM),
        ],
        out_specs=pl.BlockSpec(memory_space=pltpu.VMEM),
        scratch_shapes=[
            pltpu.VMEM((half, n), jnp.bfloat16),
            pltpu.VMEM((half, n), jnp.bfloat16),
            pltpu.VMEM((half, n), jnp.bfloat16),
            pltpu.VMEM((half, n), jnp.bfloat16),
            pltpu.SemaphoreType.DMA((C,)),
            pltpu.SemaphoreType.DMA((C,)),
            pltpu.SemaphoreType.DMA((C,)),
            pltpu.SemaphoreType.DMA((C,)),
        ],
        compiler_params=pltpu.CompilerParams(collective_id=0),
    )(dy, W)


# device time: 13573 ns/iter; 1.6061x vs baseline; 1.1655x over previous
import jax
import jax.numpy as jnp
from jax import lax
from jax.experimental import pallas as pl
from jax.experimental.pallas import tpu as pltpu

C = 8


def kernel(dy, W):
    m, k = dy.shape
    n, k2 = W.shape
    assert k == k2
    half = m // 2
    ch = half // C

    my_x_out = lax.axis_index("x")
    dy_half = lax.dynamic_slice(dy, (my_x_out * half, 0), (half, k))

    def body(dyh_ref, w_ref, out_ref, pbuf, ybuf, sbuf, xbuf,
             ysend_sems, yrecv_sems, xsend_sems, xrecv_sems):
        my_x = lax.axis_index("x")
        my_y = lax.axis_index("y")
        ynbr = (my_x, 1 - my_y)
        xnbr = (1 - my_x, my_y)
        row0 = my_x * half
        other0 = (1 - my_x) * half

        wbf = w_ref[...].astype(jnp.bfloat16)
        dybf = dyh_ref[...].astype(jnp.bfloat16)
        pm = lax.dot_general(
            dybf, wbf,
            dimension_numbers=(((1,), (1,)), ((), ())),
            preferred_element_type=jnp.float32,
        )
        pbuf[...] = pm.astype(jnp.bfloat16)

        barrier_sem = pltpu.get_barrier_semaphore()
        for nbr in (ynbr, xnbr):
            pl.semaphore_signal(
                barrier_sem, inc=1, device_id=nbr,
                device_id_type=pl.DeviceIdType.MESH,
            )
        pl.semaphore_wait(barrier_sem, 2)

        y_rdmas = []
        for c in range(C):
            r = pl.ds(c * ch, ch)
            rdma = pltpu.make_async_remote_copy(
                src_ref=pbuf.at[r, :],
                dst_ref=ybuf.at[r, :],
                send_sem=ysend_sems.at[c],
                recv_sem=yrecv_sems.at[c],
                device_id=ynbr,
                device_id_type=pl.DeviceIdType.MESH,
            )
            rdma.start()
            y_rdmas.append(rdma)

        x_rdmas = []
        for c in range(C):
            y_rdmas[c].wait_recv()
            r = pl.ds(c * ch, ch)
            sum_c = pm[c * ch:(c + 1) * ch, :] + ybuf[r, :].astype(jnp.float32)
            out_ref[pl.ds(row0 + c * ch, ch), :] = sum_c
            sbuf[r, :] = sum_c.astype(jnp.bfloat16)
            rdma = pltpu.make_async_remote_copy(
                src_ref=sbuf.at[r, :],
                dst_ref=xbuf.at[r, :],
                send_sem=xsend_sems.at[c],
                recv_sem=xrecv_sems.at[c],
                device_id=xnbr,
                device_id_type=pl.DeviceIdType.MESH,
            )
            rdma.start()
            x_rdmas.append(rdma)

        for c in range(C):
            x_rdmas[c].wait_recv()
            r = pl.ds(c * ch, ch)
            out_ref[pl.ds(other0 + c * ch, ch), :] = (
                xbuf[r, :].astype(jnp.float32)
            )

        for c in range(C):
            y_rdmas[c].wait_send()
            x_rdmas[c].wait_send()

    return pl.pallas_call(
        body,
        out_shape=jax.ShapeDtypeStruct((m, n), jnp.float32),
        in_specs=[
            pl.BlockSpec(memory_space=pltpu.VMEM),
            pl.BlockSpec(memory_space=pltpu.VMEM),
        ],
        out_specs=pl.BlockSpec(memory_space=pltpu.VMEM),
        scratch_shapes=[
            pltpu.VMEM((half, n), jnp.bfloat16),
            pltpu.VMEM((half, n), jnp.bfloat16),
            pltpu.VMEM((half, n), jnp.bfloat16),
            pltpu.VMEM((half, n), jnp.bfloat16),
            pltpu.SemaphoreType.DMA((C,)),
            pltpu.SemaphoreType.DMA((C,)),
            pltpu.SemaphoreType.DMA((C,)),
            pltpu.SemaphoreType.DMA((C,)),
        ],
        compiler_params=pltpu.CompilerParams(collective_id=0),
    )(dy_half, W)
